# baseline (speedup 1.0000x reference)
import jax
import jax.numpy as jnp
from jax import lax
from jax.experimental import pallas as pl
from jax.experimental.pallas import tpu as pltpu
from jax.experimental.pallas import tpu_sc as plsc

BATCH = 4096
SEQ = 200
D = 64
NC = 2
NS = 16
NW = NC * NS
ROWS = BATCH * SEQ
RPW = ROWS // NW
CHUNKS = RPW // SEQ
G1 = 104
G2 = SEQ - G1
LANES = 16
NBUF = 4


def _sc_body(idx_h, table_h, pe_h, out_h, idx_v, pe_v,
             b0, b1, b2, b3, g0, g1s, g2s, g3s, w0, w1, w2, w3):
    bufs = (b0, b1, b2, b3)
    gsem = (g0, g1s, g2s, g3s)
    wsem = (w0, w1, w2, w3)

    wid = lax.axis_index("s") * NC + lax.axis_index("c")
    base = wid * RPW

    pltpu.sync_copy(idx_h.at[pl.ds(base, RPW)], idx_v)
    pltpu.sync_copy(pe_h, pe_v)

    def issue_gather(c, b):
        row0 = c * SEQ
        pltpu.async_copy(table_h.at[idx_v.at[pl.ds(row0, G1)]],
                         bufs[b].at[pl.ds(0, G1)], gsem[b])
        pltpu.async_copy(table_h.at[idx_v.at[pl.ds(row0 + G1, G2)]],
                         bufs[b].at[pl.ds(G1, G2)], gsem[b])

    def wait_gather(b):
        pltpu.make_async_copy(table_h.at[idx_v.at[pl.ds(0, G1)]],
                              bufs[b].at[pl.ds(0, G1)], gsem[b]).wait()
        pltpu.make_async_copy(table_h.at[idx_v.at[pl.ds(0, G2)]],
                              bufs[b].at[pl.ds(G1, G2)], gsem[b]).wait()

    def issue_wb(c, b):
        pltpu.async_copy(bufs[b], out_h.at[pl.ds(base + c * SEQ, SEQ)],
                         wsem[b])

    def wait_wb(b):
        pltpu.make_async_copy(bufs[b], out_h.at[pl.ds(base, SEQ)],
                              wsem[b]).wait()

    def add_pe(b):
        buf = bufs[b]

        @pl.loop(0, SEQ, unroll=8)
        def _row(r):
            for j in range(D // LANES):
                sl = pl.ds(j * LANES, LANES)
                buf[r, sl] = buf[r, sl] + pe_v[r, sl]

    def slot(c, b, wait_free, pf):
        wait_gather(b)
        add_pe(b)
        issue_wb(c, b)
        if pf:
            b2 = (b + 2) % NBUF
            if wait_free:
                wait_wb(b2)
            issue_gather(c + 2, b2)

    issue_gather(0, 0)
    issue_gather(1, 1)

    slot(0, 0, False, True)
    slot(1, 1, False, True)
    slot(2, 2, True, True)
    slot(3, 3, True, True)

    @pl.loop(1, CHUNKS // NBUF - 1)
    def _group(g):
        c0 = g * NBUF
        for b in range(NBUF):
            slot(c0 + b, b, True, True)

    slot(CHUNKS - 4, 0, True, True)
    slot(CHUNKS - 3, 1, True, True)
    slot(CHUNKS - 2, 2, True, False)
    slot(CHUNKS - 1, 3, True, False)

    for b in range(NBUF):
        wait_wb(b)


@jax.jit
def _run(x_flat, table, pe_seq):
    mesh = plsc.VectorSubcoreMesh(
        core_axis_name="c", subcore_axis_name="s", num_cores=NC,
        num_subcores=NS)
    grid_kernel = pl.kernel(
        _sc_body,
        out_type=jax.ShapeDtypeStruct((ROWS, D), jnp.float32),
        mesh=mesh,
        scratch_types=(
            [pltpu.VMEM((RPW,), jnp.int32),
             pltpu.VMEM((SEQ, D), jnp.float32)]
            + [pltpu.VMEM((SEQ, D), jnp.float32) for _ in range(NBUF)]
            + [pltpu.SemaphoreType.DMA for _ in range(2 * NBUF)]
        ),
        compiler_params=pltpu.CompilerParams(use_tc_tiling_on_sc=False),
    )
    return grid_kernel(x_flat, table, pe_seq)


def kernel(x, table, pe):
    x_flat = x.reshape(ROWS)
    out = _run(x_flat, table, pe[:SEQ])
    return out.reshape(BATCH, SEQ, D)
